# baseline (speedup 1.0000x reference)
import jax
import jax.numpy as jnp
from jax.experimental import pallas as pl
from functools import partial


def _gat_dense_body(x_ref, w_ref, asrc_ref, adst_ref, xw_ref, as_ref, ad_ref,
                    *, heads, out_ch):
    x = x_ref[...]
    w = w_ref[...]
    xw = jnp.dot(x, w, preferred_element_type=jnp.float32)
    xw_ref[...] = xw
    b = xw.shape[0]
    xwh = xw.reshape(b, heads, out_ch)
    as_ref[...] = (xwh * asrc_ref[...]).sum(-1)
    ad_ref[...] = (xwh * adst_ref[...]).sum(-1)


def _gat_dense(x, W, a_src, a_dst, heads, out_ch, block_n=1000):
    n, in_ch = x.shape
    hc = heads * out_ch
    grid = (n // block_n,)
    out_shapes = (
        jax.ShapeDtypeStruct((n, hc), jnp.float32),
        jax.ShapeDtypeStruct((n, heads), jnp.float32),
        jax.ShapeDtypeStruct((n, heads), jnp.float32),
    )
    return pl.pallas_call(
        partial(_gat_dense_body, heads=heads, out_ch=out_ch),
        grid=grid,
        in_specs=[
            pl.BlockSpec((block_n, in_ch), lambda i: (i, 0)),
            pl.BlockSpec((in_ch, hc), lambda i: (0, 0)),
            pl.BlockSpec((1, heads, out_ch), lambda i: (0, 0, 0)),
            pl.BlockSpec((1, heads, out_ch), lambda i: (0, 0, 0)),
        ],
        out_specs=(
            pl.BlockSpec((block_n, hc), lambda i: (i, 0)),
            pl.BlockSpec((block_n, heads), lambda i: (i, 0)),
            pl.BlockSpec((block_n, heads), lambda i: (i, 0)),
        ),
        out_shape=out_shapes,
    )(x, W, a_src, a_dst)


def _gat_layer(x, src, dst, W, a_src, a_dst, bias, heads, out_ch, concat, n):
    xw, alpha_src, alpha_dst = _gat_dense(x, W, a_src, a_dst, heads, out_ch)
    alpha = alpha_src[src] + alpha_dst[dst]
    alpha = jax.nn.leaky_relu(alpha, negative_slope=0.2)
    amax = jax.ops.segment_max(alpha, dst, num_segments=n)
    amax = jnp.where(jnp.isfinite(amax), amax, 0.0)
    ex = jnp.exp(alpha - amax[dst])
    denom = jax.ops.segment_sum(ex, dst, num_segments=n)
    attn = ex / (denom[dst] + 1e-16)
    xwh = xw.reshape(n, heads, out_ch)
    msg = xwh[src] * attn[:, :, None]
    out = jax.ops.segment_sum(msg, dst, num_segments=n)
    if concat:
        out = out.reshape(n, heads * out_ch)
    else:
        out = out.mean(axis=1)
    return out + bias


def kernel(x, edge_index, W1, a_src1, a_dst1, b1, W2, a_src2, a_dst2, b2):
    n = x.shape[0]
    loop = jnp.arange(n, dtype=edge_index.dtype)
    src = jnp.concatenate([edge_index[0], loop])
    dst = jnp.concatenate([edge_index[1], loop])
    heads1 = a_src1.shape[1]
    hid1 = a_src1.shape[2]
    heads2 = a_src2.shape[1]
    hid2 = a_src2.shape[2]
    h = _gat_layer(x, src, dst, W1, a_src1, a_dst1, b1, heads1, hid1, True, n)
    h = jax.nn.elu(h)
    out = _gat_layer(h, src, dst, W2, a_src2, a_dst2, b2, heads2, hid2, False, n)
    return out
